# baseline (speedup 1.0000x reference)
import jax
import jax.numpy as jnp
from jax.experimental import pallas as pl
from jax.experimental.pallas import tpu as pltpu

_B, _NC, _NF, _CSKIP, _H, _K, _G = 4, 2048, 8192, 128, 128, 3, 8
_T = 512
_NT = _NF // _T
_TM = 1024
_NTM = _NF // _TM
_GN_N = (_H // _G) * _NF
_KEEP = ~2047


def _dot(a, b):
    return jax.lax.dot_general(a, b, (((1,), (0,)), ((), ())),
                               preferred_element_type=jnp.float32)


def _dotb(a, b):
    return _dot(a.astype(jnp.bfloat16), b.astype(jnp.bfloat16))


def _silu(x):
    return x / (1.0 + jnp.exp(-x))


def _knn_body(pct_ref, pcb_ref, pfb_ref, pf_ref, dpc_ref, fs_ref,
              w1a_ref, w1b_ref,
              dp3_ref, h1_ref, s1_ref, q1_ref, pp_ref):
    t = pl.program_id(1)

    @pl.when(t == 0)
    def _():
        pc = pct_ref[0]
        pp_ref[...] = (pc[:, 0:1] * pc[:, 0:1] + pc[:, 1:2] * pc[:, 1:2]
                       + pc[:, 2:3] * pc[:, 2:3])

    pf = pf_ref[0]
    qq = pf[0:1] * pf[0:1] + pf[1:2] * pf[1:2] + pf[2:3] * pf[2:3]
    prod = _dot(pcb_ref[0], pfb_ref[0])
    d2 = (pp_ref[...] - 2.0 * prod) + qq

    ibits = jax.lax.broadcasted_iota(jnp.int32, (_NC, _T), 0)
    key = jax.lax.bitcast_convert_type(
        (jax.lax.bitcast_convert_type(d2, jnp.int32) & _KEEP) | ibits,
        jnp.float32)
    inf = jnp.inf
    m1 = jnp.min(key, axis=0, keepdims=True)
    m2 = jnp.min(jnp.where(key > m1, key, inf), axis=0, keepdims=True)
    m3 = jnp.min(jnp.where(key > m2, key, inf), axis=0, keepdims=True)
    wsum = (1.0 / jnp.maximum(m1, 1e-12)
            + 1.0 / jnp.maximum(m2, 1e-12)
            + 1.0 / jnp.maximum(m3, 1e-12))
    wmat = jnp.where(key <= m3,
                     1.0 / jnp.maximum(key, 1e-12), 0.0).astype(jnp.bfloat16)

    dp3 = _dot(dpc_ref[0], wmat) / wsum
    h1 = _dotb(w1a_ref[...], dp3) + _dot(w1b_ref[...], fs_ref[0])
    dp3_ref[0] = dp3
    h1_ref[0] = h1.astype(jnp.bfloat16)

    @pl.when(t == 0)
    def _():
        s1_ref[...] = jnp.zeros_like(s1_ref)
        q1_ref[...] = jnp.zeros_like(q1_ref)

    s1_ref[0] += jnp.sum(h1, axis=1, keepdims=True)
    q1_ref[0] += jnp.sum(h1 * h1, axis=1, keepdims=True)


def _gn_affine(s_ref, q_ref, g_ref, b_ref):
    r = jax.lax.broadcasted_iota(jnp.int32, (_H, _H), 0) // (_H // _G)
    c = jax.lax.broadcasted_iota(jnp.int32, (_H, _H), 1) // (_H // _G)
    A = (r == c).astype(jnp.float32)
    mean = _dot(A, s_ref[0]) * (1.0 / _GN_N)
    var = _dot(A, q_ref[0]) * (1.0 / _GN_N) - mean * mean
    inv = jax.lax.rsqrt(var + 1e-5)
    a = g_ref[...] * inv
    return a, b_ref[...] - mean * a


def _mid_body(h1_ref, s1_ref, q1_ref, g_ref, b_ref, w2_ref,
              h2_ref, s2_ref, q2_ref):
    t = pl.program_id(1)
    a, c = _gn_affine(s1_ref, q1_ref, g_ref, b_ref)
    act = _silu(h1_ref[0].astype(jnp.float32) * a + c)
    h2 = _dotb(w2_ref[...], act)
    h2_ref[0] = h2

    @pl.when(t == 0)
    def _():
        s2_ref[...] = jnp.zeros_like(s2_ref)
        q2_ref[...] = jnp.zeros_like(q2_ref)

    s2_ref[0] += jnp.sum(h2, axis=1, keepdims=True)
    q2_ref[0] += jnp.sum(h2 * h2, axis=1, keepdims=True)


def _out_body(h2_ref, s2_ref, q2_ref, g_ref, b_ref, w3_ref, b3_ref, dp3_ref,
              out_ref):
    a, c = _gn_affine(s2_ref, q2_ref, g_ref, b_ref)
    act = _silu(h2_ref[0].astype(jnp.float32) * a + c)
    out_ref[0] = dp3_ref[0] + _dotb(w3_ref[...], act) + b3_ref[...]


def kernel(P_coarse_b3n, P_fine_b3n, dP_coarse_b3n, F_skip_bcn, W1, g1, b1,
           W2, g2, b2, W3, b3):
    f32 = jnp.float32
    bf16 = jnp.bfloat16
    pct = jnp.transpose(P_coarse_b3n, (0, 2, 1))
    pcb = pct.astype(bf16)
    pfb = P_fine_b3n.astype(bf16)
    dpcb = dP_coarse_b3n.astype(bf16)
    fsb = F_skip_bcn.astype(bf16)
    w1a = W1[:, :3]
    w1b = W1[:, 3:].astype(bf16)
    g1c, b1c = g1.reshape(_H, 1), b1.reshape(_H, 1)
    g2c, b2c = g2.reshape(_H, 1), b2.reshape(_H, 1)
    b3c = b3.reshape(3, 1)

    arb = pltpu.CompilerParams(
        dimension_semantics=("arbitrary", "arbitrary"))

    full = lambda shape: pl.BlockSpec(shape, lambda bi, ti: (0,) * len(shape))
    perb = lambda shape: pl.BlockSpec(shape, lambda bi, ti: (bi,) + (0,) * (len(shape) - 1))
    tile = lambda shape: pl.BlockSpec(shape, lambda bi, ti: (bi, 0, ti))

    dp3, h1, s1, q1 = pl.pallas_call(
        _knn_body,
        grid=(_B, _NT),
        in_specs=[perb((1, _NC, 3)), perb((1, _NC, 3)), tile((1, 3, _T)),
                  tile((1, 3, _T)), perb((1, 3, _NC)), tile((1, _CSKIP, _T)),
                  full((_H, 3)), full((_H, _CSKIP))],
        out_specs=[tile((1, 3, _T)), tile((1, _H, _T)),
                   perb((1, _H, 1)), perb((1, _H, 1))],
        out_shape=[jax.ShapeDtypeStruct((_B, 3, _NF), f32),
                   jax.ShapeDtypeStruct((_B, _H, _NF), bf16),
                   jax.ShapeDtypeStruct((_B, _H, 1), f32),
                   jax.ShapeDtypeStruct((_B, _H, 1), f32)],
        scratch_shapes=[pltpu.VMEM((_NC, 1), f32)],
        compiler_params=arb,
    )(pct, pcb, pfb, P_fine_b3n, dpcb, fsb, w1a, w1b)

    h2, s2, q2 = pl.pallas_call(
        _mid_body,
        grid=(_B, _NTM),
        in_specs=[tile((1, _H, _TM)), perb((1, _H, 1)), perb((1, _H, 1)),
                  full((_H, 1)), full((_H, 1)), full((_H, _H))],
        out_specs=[tile((1, _H, _TM)), perb((1, _H, 1)), perb((1, _H, 1))],
        out_shape=[jax.ShapeDtypeStruct((_B, _H, _NF), f32),
                   jax.ShapeDtypeStruct((_B, _H, 1), f32),
                   jax.ShapeDtypeStruct((_B, _H, 1), f32)],
        compiler_params=arb,
    )(h1, s1, q1, g1c, b1c, W2)

    out = pl.pallas_call(
        _out_body,
        grid=(_B, _NTM),
        in_specs=[tile((1, _H, _TM)), perb((1, _H, 1)), perb((1, _H, 1)),
                  full((_H, 1)), full((_H, 1)), full((3, _H)), full((3, 1)),
                  tile((1, 3, _TM))],
        out_specs=tile((1, 3, _TM)),
        out_shape=jax.ShapeDtypeStruct((_B, 3, _NF), f32),
        compiler_params=arb,
    )(h2, s2, q2, g2c, b2c, W3, b3c, dp3)

    return out


# speedup vs baseline: 1.0183x; 1.0183x over previous
import functools

import jax
import jax.numpy as jnp
from jax import lax
from jax.experimental import pallas as pl
from jax.experimental.pallas import tpu as pltpu
from jax.experimental.pallas import tpu_sc as plsc

_B, _NC, _NF, _CSKIP, _H, _K, _G = 4, 2048, 8192, 128, 128, 3, 8
_T = 512
_NT = _NF // _T
_TM = 1024
_NTM = _NF // _TM
_GN_N = (_H // _G) * _NF
_KEEP = ~2047
_NW = 32
_PW = _B * _NF // _NW
_CPB = _NF // _PW


def _dot(a, b):
    return jax.lax.dot_general(a, b, (((1,), (0,)), ((), ())),
                               preferred_element_type=jnp.float32)


def _dotb(a, b):
    return _dot(a.astype(jnp.bfloat16), b.astype(jnp.bfloat16))


def _silu(x):
    return x / (1.0 + jnp.exp(-x))


def _knn_body(pct_ref, pcb_ref, pfb_ref, pf_ref, idx_ref, wn_ref, pp_ref):
    t = pl.program_id(1)

    @pl.when(t == 0)
    def _():
        pc = pct_ref[0]
        pp_ref[...] = (pc[:, 0:1] * pc[:, 0:1] + pc[:, 1:2] * pc[:, 1:2]
                       + pc[:, 2:3] * pc[:, 2:3])

    pf = pf_ref[0]
    qq = pf[0:1] * pf[0:1] + pf[1:2] * pf[1:2] + pf[2:3] * pf[2:3]
    prod = _dot(pcb_ref[0], pfb_ref[0])
    d2 = (pp_ref[...] - 2.0 * prod) + qq

    ibits = jax.lax.broadcasted_iota(jnp.int32, (_NC, _T), 0)
    key = jax.lax.bitcast_convert_type(
        (jax.lax.bitcast_convert_type(d2, jnp.int32) & _KEEP) | ibits,
        jnp.float32)
    inf = jnp.inf
    m1 = jnp.min(key, axis=0, keepdims=True)
    m2 = jnp.min(jnp.where(key > m1, key, inf), axis=0, keepdims=True)
    m3 = jnp.min(jnp.where(key > m2, key, inf), axis=0, keepdims=True)
    w1 = 1.0 / jnp.maximum(m1, 1e-12)
    w2 = 1.0 / jnp.maximum(m2, 1e-12)
    w3 = 1.0 / jnp.maximum(m3, 1e-12)
    rs = 1.0 / (w1 + w2 + w3)
    idx_ref[0] = jnp.concatenate(
        [jax.lax.bitcast_convert_type(m, jnp.int32) & 2047
         for m in (m1, m2, m3)], axis=0)
    wn_ref[0] = jnp.concatenate([w1 * rs, w2 * rs, w3 * rs], axis=0)


def _sc_gather_body(idx_hbm, wn_hbm, dp_hbm, out_hbm, idx_v, w_v, tab_v,
                    out_v):
    wid = lax.axis_index("s") * 2 + lax.axis_index("c")
    b = wid // _CPB
    c = wid % _CPB
    sl_hbm = pl.ds(c * _PW, _PW)
    pltpu.sync_copy(dp_hbm.at[b], tab_v)
    pltpu.sync_copy(idx_hbm.at[b, :, sl_hbm], idx_v)
    pltpu.sync_copy(wn_hbm.at[b, :, sl_hbm], w_v)
    def body(j, carry):
        sl = pl.ds(j * 16, 16)
        for d in range(3):
            acc = jnp.zeros((16,), jnp.float32)
            for k in range(3):
                g = plsc.load_gather(tab_v, [idx_v[k, sl] + d * _NC])
                acc = acc + w_v[k, sl] * g
            out_v[d, sl] = acc
        return carry

    lax.fori_loop(0, _PW // 16, body, 0)
    pltpu.sync_copy(out_v, out_hbm.at[b, :, sl_hbm])


def _h1_body(dp3_ref, fs_ref, w1a_ref, w1b_ref, h1_ref, s1_ref, q1_ref):
    t = pl.program_id(1)
    h1 = _dotb(w1a_ref[...], dp3_ref[0]) + _dot(w1b_ref[...], fs_ref[0])
    h1_ref[0] = h1.astype(jnp.bfloat16)

    @pl.when(t == 0)
    def _():
        s1_ref[...] = jnp.zeros_like(s1_ref)
        q1_ref[...] = jnp.zeros_like(q1_ref)

    s1_ref[0] += jnp.sum(h1, axis=1, keepdims=True)
    q1_ref[0] += jnp.sum(h1 * h1, axis=1, keepdims=True)


def _gn_affine(s_ref, q_ref, g_ref, b_ref):
    r = jax.lax.broadcasted_iota(jnp.int32, (_H, _H), 0) // (_H // _G)
    c = jax.lax.broadcasted_iota(jnp.int32, (_H, _H), 1) // (_H // _G)
    A = (r == c).astype(jnp.float32)
    mean = _dot(A, s_ref[0]) * (1.0 / _GN_N)
    var = _dot(A, q_ref[0]) * (1.0 / _GN_N) - mean * mean
    inv = jax.lax.rsqrt(var + 1e-5)
    a = g_ref[...] * inv
    return a, b_ref[...] - mean * a


def _mid_body(h1_ref, s1_ref, q1_ref, g_ref, b_ref, w2_ref,
              h2_ref, s2_ref, q2_ref):
    t = pl.program_id(1)
    a, c = _gn_affine(s1_ref, q1_ref, g_ref, b_ref)
    act = _silu(h1_ref[0].astype(jnp.float32) * a + c)
    h2 = _dotb(w2_ref[...], act)
    h2_ref[0] = h2

    @pl.when(t == 0)
    def _():
        s2_ref[...] = jnp.zeros_like(s2_ref)
        q2_ref[...] = jnp.zeros_like(q2_ref)

    s2_ref[0] += jnp.sum(h2, axis=1, keepdims=True)
    q2_ref[0] += jnp.sum(h2 * h2, axis=1, keepdims=True)


def _out_body(h2_ref, s2_ref, q2_ref, g_ref, b_ref, w3_ref, b3_ref, dp3_ref,
              out_ref):
    a, c = _gn_affine(s2_ref, q2_ref, g_ref, b_ref)
    act = _silu(h2_ref[0].astype(jnp.float32) * a + c)
    out_ref[0] = dp3_ref[0] + _dotb(w3_ref[...], act) + b3_ref[...]


def kernel(P_coarse_b3n, P_fine_b3n, dP_coarse_b3n, F_skip_bcn, W1, g1, b1,
           W2, g2, b2, W3, b3):
    f32 = jnp.float32
    bf16 = jnp.bfloat16
    i32 = jnp.int32
    pct = jnp.transpose(P_coarse_b3n, (0, 2, 1))
    pcb = pct.astype(bf16)
    pfb = P_fine_b3n.astype(bf16)
    fsb = F_skip_bcn.astype(bf16)
    w1a = W1[:, :3]
    w1b = W1[:, 3:].astype(bf16)
    g1c, b1c = g1.reshape(_H, 1), b1.reshape(_H, 1)
    g2c, b2c = g2.reshape(_H, 1), b2.reshape(_H, 1)
    b3c = b3.reshape(3, 1)

    arb = pltpu.CompilerParams(
        dimension_semantics=("arbitrary", "arbitrary"))

    full = lambda shape: pl.BlockSpec(shape, lambda bi, ti: (0,) * len(shape))
    perb = lambda shape: pl.BlockSpec(shape, lambda bi, ti: (bi,) + (0,) * (len(shape) - 1))
    tile = lambda shape: pl.BlockSpec(shape, lambda bi, ti: (bi, 0, ti))

    idx, wn = pl.pallas_call(
        _knn_body,
        grid=(_B, _NT),
        in_specs=[perb((1, _NC, 3)), perb((1, _NC, 3)), tile((1, 3, _T)),
                  tile((1, 3, _T))],
        out_specs=[tile((1, 3, _T)), tile((1, 3, _T))],
        out_shape=[jax.ShapeDtypeStruct((_B, 3, _NF), i32),
                   jax.ShapeDtypeStruct((_B, 3, _NF), f32)],
        scratch_shapes=[pltpu.VMEM((_NC, 1), f32)],
        compiler_params=arb,
    )(pct, pcb, pfb, P_fine_b3n)

    mesh = plsc.VectorSubcoreMesh(core_axis_name="c", subcore_axis_name="s")
    dp3 = functools.partial(
        pl.kernel,
        out_type=jax.ShapeDtypeStruct((_B, 3, _NF), f32),
        mesh=mesh,
        compiler_params=pltpu.CompilerParams(needs_layout_passes=False),
        scratch_types=[pltpu.VMEM((3, _PW), i32), pltpu.VMEM((3, _PW), f32),
                       pltpu.VMEM((3 * _NC,), f32), pltpu.VMEM((3, _PW), f32)],
    )(_sc_gather_body)(idx, wn, dP_coarse_b3n.reshape(_B, 3 * _NC))

    h1, s1, q1 = pl.pallas_call(
        _h1_body,
        grid=(_B, _NT),
        in_specs=[tile((1, 3, _T)), tile((1, _CSKIP, _T)),
                  full((_H, 3)), full((_H, _CSKIP))],
        out_specs=[tile((1, _H, _T)), perb((1, _H, 1)), perb((1, _H, 1))],
        out_shape=[jax.ShapeDtypeStruct((_B, _H, _NF), bf16),
                   jax.ShapeDtypeStruct((_B, _H, 1), f32),
                   jax.ShapeDtypeStruct((_B, _H, 1), f32)],
        compiler_params=arb,
    )(dp3, fsb, w1a, w1b)

    h2, s2, q2 = pl.pallas_call(
        _mid_body,
        grid=(_B, _NTM),
        in_specs=[tile((1, _H, _TM)), perb((1, _H, 1)), perb((1, _H, 1)),
                  full((_H, 1)), full((_H, 1)), full((_H, _H))],
        out_specs=[tile((1, _H, _TM)), perb((1, _H, 1)), perb((1, _H, 1))],
        out_shape=[jax.ShapeDtypeStruct((_B, _H, _NF), f32),
                   jax.ShapeDtypeStruct((_B, _H, 1), f32),
                   jax.ShapeDtypeStruct((_B, _H, 1), f32)],
        compiler_params=arb,
    )(h1, s1, q1, g1c, b1c, W2)

    out = pl.pallas_call(
        _out_body,
        grid=(_B, _NTM),
        in_specs=[tile((1, _H, _TM)), perb((1, _H, 1)), perb((1, _H, 1)),
                  full((_H, 1)), full((_H, 1)), full((3, _H)), full((3, 1)),
                  tile((1, 3, _TM))],
        out_specs=tile((1, 3, _TM)),
        out_shape=jax.ShapeDtypeStruct((_B, 3, _NF), f32),
        compiler_params=arb,
    )(h2, s2, q2, g2c, b2c, W3, b3c, dp3)

    return out


# speedup vs baseline: 1.1777x; 1.1565x over previous
import functools

import jax
import jax.numpy as jnp
from jax import lax
from jax.experimental import pallas as pl
from jax.experimental.pallas import tpu as pltpu
from jax.experimental.pallas import tpu_sc as plsc

_B, _NC, _NF, _CSKIP, _H, _K, _G = 4, 2048, 8192, 128, 128, 3, 8
_T = 1024
_NT = _NF // _T
_TM = 1024
_NTM = _NF // _TM
_GN_N = (_H // _G) * _NF
_KEEP = ~2047
_NW = 32
_PW = _B * _NF // _NW
_CPB = _NF // _PW


def _dot(a, b):
    return jax.lax.dot_general(a, b, (((1,), (0,)), ((), ())),
                               preferred_element_type=jnp.float32)


def _dotb(a, b):
    return _dot(a.astype(jnp.bfloat16), b.astype(jnp.bfloat16))


def _silu(x):
    return x / (1.0 + jnp.exp(-x))


def _knn_body(pct_ref, pcb_ref, pfb_ref, pf_ref, idx_ref, wn_ref, pc7_ref):
    t = pl.program_id(1)
    f32, bf16 = jnp.float32, jnp.bfloat16

    @pl.when(t == 0)
    def _():
        pc = pct_ref[0]
        pp = (pc[:, 0:1] * pc[:, 0:1] + pc[:, 1:2] * pc[:, 1:2]
              + pc[:, 2:3] * pc[:, 2:3])
        pph = pp.astype(bf16).astype(f32)
        ppl = pp - pph
        one = jnp.ones_like(pp)
        pc7_ref[...] = jnp.concatenate(
            [-2.0 * pcb_ref[0].astype(f32), pph, ppl, one, one], axis=1)

    pf = pf_ref[0]
    qq = pf[0:1] * pf[0:1] + pf[1:2] * pf[1:2] + pf[2:3] * pf[2:3]
    qqh = qq.astype(bf16).astype(f32)
    qql = qq - qqh
    one = jnp.ones_like(qq)
    pf7 = jnp.concatenate(
        [pfb_ref[0].astype(f32), one, one, qqh, qql], axis=0)
    d2 = _dotb(pc7_ref[...], pf7)

    ibits = jax.lax.broadcasted_iota(jnp.int32, (_NC, _T), 0)
    key = jax.lax.bitcast_convert_type(
        (jax.lax.bitcast_convert_type(d2, jnp.int32) & _KEEP) | ibits,
        jnp.float32)
    inf = jnp.inf
    m1 = jnp.min(key, axis=0, keepdims=True)
    m2 = jnp.min(jnp.where(key > m1, key, inf), axis=0, keepdims=True)
    m3 = jnp.min(jnp.where(key > m2, key, inf), axis=0, keepdims=True)
    w1 = 1.0 / jnp.maximum(m1, 1e-12)
    w2 = 1.0 / jnp.maximum(m2, 1e-12)
    w3 = 1.0 / jnp.maximum(m3, 1e-12)
    rs = 1.0 / (w1 + w2 + w3)
    idx_ref[0] = jnp.concatenate(
        [jax.lax.bitcast_convert_type(m, jnp.int32) & 2047
         for m in (m1, m2, m3)], axis=0)
    wn_ref[0] = jnp.concatenate([w1 * rs, w2 * rs, w3 * rs], axis=0)


def _sc_gather_body(idx_hbm, wn_hbm, dp_hbm, out_hbm, idx_v, w_v, tab_v,
                    out_v):
    wid = lax.axis_index("s") * 2 + lax.axis_index("c")
    b = wid // _CPB
    c = wid % _CPB
    sl_hbm = pl.ds(c * _PW, _PW)
    pltpu.sync_copy(dp_hbm.at[b], tab_v)
    pltpu.sync_copy(idx_hbm.at[b, :, sl_hbm], idx_v)
    pltpu.sync_copy(wn_hbm.at[b, :, sl_hbm], w_v)
    def body(j, carry):
        sl = pl.ds(j * 16, 16)
        for d in range(3):
            acc = jnp.zeros((16,), jnp.float32)
            for k in range(3):
                g = plsc.load_gather(tab_v, [idx_v[k, sl] + d * _NC])
                acc = acc + w_v[k, sl] * g
            out_v[d, sl] = acc
        return carry

    lax.fori_loop(0, _PW // 16, body, 0)
    pltpu.sync_copy(out_v, out_hbm.at[b, :, sl_hbm])


def _h1_body(dp3_ref, fs_ref, w1a_ref, w1b_ref, h1_ref, s1_ref, q1_ref):
    t = pl.program_id(1)
    h1 = _dotb(w1a_ref[...], dp3_ref[0]) + _dot(w1b_ref[...], fs_ref[0])
    h1_ref[0] = h1.astype(jnp.bfloat16)

    @pl.when(t == 0)
    def _():
        s1_ref[...] = jnp.zeros_like(s1_ref)
        q1_ref[...] = jnp.zeros_like(q1_ref)

    s1_ref[0] += jnp.sum(h1, axis=1, keepdims=True)
    q1_ref[0] += jnp.sum(h1 * h1, axis=1, keepdims=True)


def _gn_affine(s_ref, q_ref, g_ref, b_ref):
    r = jax.lax.broadcasted_iota(jnp.int32, (_H, _H), 0) // (_H // _G)
    c = jax.lax.broadcasted_iota(jnp.int32, (_H, _H), 1) // (_H // _G)
    A = (r == c).astype(jnp.float32)
    mean = _dot(A, s_ref[0]) * (1.0 / _GN_N)
    var = _dot(A, q_ref[0]) * (1.0 / _GN_N) - mean * mean
    inv = jax.lax.rsqrt(var + 1e-5)
    a = g_ref[...] * inv
    return a, b_ref[...] - mean * a


def _mid_body(h1_ref, s1_ref, q1_ref, g_ref, b_ref, w2_ref,
              h2_ref, s2_ref, q2_ref):
    t = pl.program_id(1)
    a, c = _gn_affine(s1_ref, q1_ref, g_ref, b_ref)
    act = _silu(h1_ref[0].astype(jnp.float32) * a + c)
    h2 = _dotb(w2_ref[...], act)
    h2_ref[0] = h2

    @pl.when(t == 0)
    def _():
        s2_ref[...] = jnp.zeros_like(s2_ref)
        q2_ref[...] = jnp.zeros_like(q2_ref)

    s2_ref[0] += jnp.sum(h2, axis=1, keepdims=True)
    q2_ref[0] += jnp.sum(h2 * h2, axis=1, keepdims=True)


def _out_body(h2_ref, s2_ref, q2_ref, g_ref, b_ref, w3_ref, b3_ref, dp3_ref,
              out_ref):
    a, c = _gn_affine(s2_ref, q2_ref, g_ref, b_ref)
    act = _silu(h2_ref[0].astype(jnp.float32) * a + c)
    out_ref[0] = dp3_ref[0] + _dotb(w3_ref[...], act) + b3_ref[...]


def kernel(P_coarse_b3n, P_fine_b3n, dP_coarse_b3n, F_skip_bcn, W1, g1, b1,
           W2, g2, b2, W3, b3):
    f32 = jnp.float32
    bf16 = jnp.bfloat16
    i32 = jnp.int32
    pct = jnp.transpose(P_coarse_b3n, (0, 2, 1))
    pcb = pct.astype(bf16)
    pfb = P_fine_b3n.astype(bf16)
    fsb = F_skip_bcn.astype(bf16)
    w1a = W1[:, :3]
    w1b = W1[:, 3:].astype(bf16)
    g1c, b1c = g1.reshape(_H, 1), b1.reshape(_H, 1)
    g2c, b2c = g2.reshape(_H, 1), b2.reshape(_H, 1)
    b3c = b3.reshape(3, 1)

    arb = pltpu.CompilerParams(
        dimension_semantics=("arbitrary", "arbitrary"))

    full = lambda shape: pl.BlockSpec(shape, lambda bi, ti: (0,) * len(shape))
    perb = lambda shape: pl.BlockSpec(shape, lambda bi, ti: (bi,) + (0,) * (len(shape) - 1))
    tile = lambda shape: pl.BlockSpec(shape, lambda bi, ti: (bi, 0, ti))

    idx, wn = pl.pallas_call(
        _knn_body,
        grid=(_B, _NT),
        in_specs=[perb((1, _NC, 3)), perb((1, _NC, 3)), tile((1, 3, _T)),
                  tile((1, 3, _T))],
        out_specs=[tile((1, 3, _T)), tile((1, 3, _T))],
        out_shape=[jax.ShapeDtypeStruct((_B, 3, _NF), i32),
                   jax.ShapeDtypeStruct((_B, 3, _NF), f32)],
        scratch_shapes=[pltpu.VMEM((_NC, 7), f32)],
        compiler_params=arb,
    )(pct, pcb, pfb, P_fine_b3n)

    mesh = plsc.VectorSubcoreMesh(core_axis_name="c", subcore_axis_name="s")
    dp3 = functools.partial(
        pl.kernel,
        out_type=jax.ShapeDtypeStruct((_B, 3, _NF), f32),
        mesh=mesh,
        compiler_params=pltpu.CompilerParams(needs_layout_passes=False),
        scratch_types=[pltpu.VMEM((3, _PW), i32), pltpu.VMEM((3, _PW), f32),
                       pltpu.VMEM((3 * _NC,), f32), pltpu.VMEM((3, _PW), f32)],
    )(_sc_gather_body)(idx, wn, dP_coarse_b3n.reshape(_B, 3 * _NC))

    h1, s1, q1 = pl.pallas_call(
        _h1_body,
        grid=(_B, _NT),
        in_specs=[tile((1, 3, _T)), tile((1, _CSKIP, _T)),
                  full((_H, 3)), full((_H, _CSKIP))],
        out_specs=[tile((1, _H, _T)), perb((1, _H, 1)), perb((1, _H, 1))],
        out_shape=[jax.ShapeDtypeStruct((_B, _H, _NF), bf16),
                   jax.ShapeDtypeStruct((_B, _H, 1), f32),
                   jax.ShapeDtypeStruct((_B, _H, 1), f32)],
        compiler_params=arb,
    )(dp3, fsb, w1a, w1b)

    h2, s2, q2 = pl.pallas_call(
        _mid_body,
        grid=(_B, _NTM),
        in_specs=[tile((1, _H, _TM)), perb((1, _H, 1)), perb((1, _H, 1)),
                  full((_H, 1)), full((_H, 1)), full((_H, _H))],
        out_specs=[tile((1, _H, _TM)), perb((1, _H, 1)), perb((1, _H, 1))],
        out_shape=[jax.ShapeDtypeStruct((_B, _H, _NF), f32),
                   jax.ShapeDtypeStruct((_B, _H, 1), f32),
                   jax.ShapeDtypeStruct((_B, _H, 1), f32)],
        compiler_params=arb,
    )(h1, s1, q1, g1c, b1c, W2)

    out = pl.pallas_call(
        _out_body,
        grid=(_B, _NTM),
        in_specs=[tile((1, _H, _TM)), perb((1, _H, 1)), perb((1, _H, 1)),
                  full((_H, 1)), full((_H, 1)), full((3, _H)), full((3, 1)),
                  tile((1, 3, _TM))],
        out_specs=tile((1, 3, _TM)),
        out_shape=jax.ShapeDtypeStruct((_B, 3, _NF), f32),
        compiler_params=arb,
    )(h2, s2, q2, g2c, b2c, W3, b3c, dp3)

    return out


# speedup vs baseline: 1.2525x; 1.0635x over previous
import functools

import jax
import jax.numpy as jnp
from jax import lax
from jax.experimental import pallas as pl
from jax.experimental.pallas import tpu as pltpu
from jax.experimental.pallas import tpu_sc as plsc

_B, _NC, _NF, _CSKIP, _H, _K, _G = 4, 2048, 8192, 128, 128, 3, 8
_T = 1024
_NT = _NF // _T
_TM = 1024
_NTM = _NF // _TM
_GN_N = (_H // _G) * _NF
_KEEP = ~2047
_NW = 32
_PW = _B * _NF // _NW
_CPB = _NF // _PW


def _dot(a, b):
    return jax.lax.dot_general(a, b, (((1,), (0,)), ((), ())),
                               preferred_element_type=jnp.float32)


def _dotb(a, b):
    return _dot(a.astype(jnp.bfloat16), b.astype(jnp.bfloat16))


def _silu(x):
    return x / (1.0 + jnp.exp(-x))


def _knn_body(pct_ref, pcb_ref, pfb_ref, pf_ref, idx_ref, wn_ref, pc7_ref):
    t = pl.program_id(1)
    f32, bf16 = jnp.float32, jnp.bfloat16

    @pl.when(t == 0)
    def _():
        pc = pct_ref[0]
        pp = (pc[:, 0:1] * pc[:, 0:1] + pc[:, 1:2] * pc[:, 1:2]
              + pc[:, 2:3] * pc[:, 2:3])
        pph = pp.astype(bf16).astype(f32)
        ppl = pp - pph
        one = jnp.ones_like(pp)
        pc7_ref[...] = jnp.concatenate(
            [-2.0 * pcb_ref[0].astype(f32), pph, ppl, one, one], axis=1)

    pf = pf_ref[0]
    qq = pf[0:1] * pf[0:1] + pf[1:2] * pf[1:2] + pf[2:3] * pf[2:3]
    qqh = qq.astype(bf16).astype(f32)
    qql = qq - qqh
    one = jnp.ones_like(qq)
    pf7 = jnp.concatenate(
        [pfb_ref[0].astype(f32), one, one, qqh, qql], axis=0)
    d2 = _dotb(pc7_ref[...], pf7)

    ibits = jax.lax.broadcasted_iota(jnp.int32, (_NC, _T), 0)
    key = jax.lax.bitcast_convert_type(
        (jax.lax.bitcast_convert_type(d2, jnp.int32) & _KEEP) | ibits,
        jnp.float32)
    inf = jnp.inf
    m1 = jnp.min(key, axis=0, keepdims=True)
    m2 = jnp.min(jnp.where(key > m1, key, inf), axis=0, keepdims=True)
    m3 = jnp.min(jnp.where(key > m2, key, inf), axis=0, keepdims=True)
    w1 = 1.0 / jnp.maximum(m1, 1e-12)
    w2 = 1.0 / jnp.maximum(m2, 1e-12)
    w3 = 1.0 / jnp.maximum(m3, 1e-12)
    rs = 1.0 / (w1 + w2 + w3)
    idx_ref[0] = jnp.concatenate(
        [jax.lax.bitcast_convert_type(m, jnp.int32) & 2047
         for m in (m1, m2, m3)], axis=0)
    wn_ref[0] = jnp.concatenate([w1 * rs, w2 * rs, w3 * rs], axis=0)


def _sc_gather_body(idx_hbm, wn_hbm, dp_hbm, out_hbm, idx_v, w_v, tab_v,
                    out_v):
    wid = lax.axis_index("s") * 2 + lax.axis_index("c")
    b = wid // _CPB
    c = wid % _CPB
    sl_hbm = pl.ds(c * _PW, _PW)
    pltpu.sync_copy(dp_hbm.at[b], tab_v)
    pltpu.sync_copy(idx_hbm.at[b, :, sl_hbm], idx_v)
    pltpu.sync_copy(wn_hbm.at[b, :, sl_hbm], w_v)
    def body(j, carry):
        sl = pl.ds(j * 16, 16)
        for d in range(3):
            acc = jnp.zeros((16,), jnp.float32)
            for k in range(3):
                g = plsc.load_gather(tab_v, [idx_v[k, sl] + d * _NC])
                acc = acc + w_v[k, sl] * g
            out_v[d, sl] = acc
        return carry

    lax.fori_loop(0, _PW // 16, body, 0)
    pltpu.sync_copy(out_v, out_hbm.at[b, :, sl_hbm])


def _gn_affine(s, q, g_ref, b_ref):
    r = jax.lax.broadcasted_iota(jnp.int32, (_H, _H), 0) // (_H // _G)
    c = jax.lax.broadcasted_iota(jnp.int32, (_H, _H), 1) // (_H // _G)
    A = (r == c).astype(jnp.float32)
    mean = _dot(A, s) * (1.0 / _GN_N)
    var = _dot(A, q) * (1.0 / _GN_N) - mean * mean
    inv = jax.lax.rsqrt(var + 1e-5)
    a = g_ref[...] * inv
    return a, b_ref[...] - mean * a


def _mlp_body(dp3_ref, fs_ref, w1a_ref, w1b_ref, g1_ref, b1_ref, w2_ref,
              g2_ref, b2_ref, w3_ref, b3_ref, out_ref,
              h1s_ref, h2s_ref, s1_ref, q1_ref, s2_ref, q2_ref):
    p = pl.program_id(1)
    t = pl.program_id(2)
    sl = pl.ds(t * _TM, _TM)

    @pl.when((p == 0) & (t == 0))
    def _():
        s1_ref[...] = jnp.zeros_like(s1_ref)
        q1_ref[...] = jnp.zeros_like(q1_ref)
        s2_ref[...] = jnp.zeros_like(s2_ref)
        q2_ref[...] = jnp.zeros_like(q2_ref)

    @pl.when(p == 0)
    def _():
        h1 = _dotb(w1a_ref[...], dp3_ref[0]) + _dot(w1b_ref[...], fs_ref[0])
        h1s_ref[:, sl] = h1.astype(jnp.bfloat16)
        s1_ref[...] += jnp.sum(h1, axis=1, keepdims=True)
        q1_ref[...] += jnp.sum(h1 * h1, axis=1, keepdims=True)

    @pl.when(p == 1)
    def _():
        a, c = _gn_affine(s1_ref[...], q1_ref[...], g1_ref, b1_ref)
        act = _silu(h1s_ref[:, sl].astype(jnp.float32) * a + c)
        h2 = _dotb(w2_ref[...], act)
        h2s_ref[:, sl] = h2
        s2_ref[...] += jnp.sum(h2, axis=1, keepdims=True)
        q2_ref[...] += jnp.sum(h2 * h2, axis=1, keepdims=True)

    @pl.when(p == 2)
    def _():
        a, c = _gn_affine(s2_ref[...], q2_ref[...], g2_ref, b2_ref)
        act = _silu(h2s_ref[:, sl] * a + c)
        out_ref[0] = dp3_ref[0] + _dotb(w3_ref[...], act) + b3_ref[...]


def kernel(P_coarse_b3n, P_fine_b3n, dP_coarse_b3n, F_skip_bcn, W1, g1, b1,
           W2, g2, b2, W3, b3):
    f32 = jnp.float32
    bf16 = jnp.bfloat16
    i32 = jnp.int32
    pct = jnp.transpose(P_coarse_b3n, (0, 2, 1))
    pcb = pct.astype(bf16)
    pfb = P_fine_b3n.astype(bf16)
    fsb = F_skip_bcn.astype(bf16)
    w1a = W1[:, :3]
    w1b = W1[:, 3:].astype(bf16)
    g1c, b1c = g1.reshape(_H, 1), b1.reshape(_H, 1)
    g2c, b2c = g2.reshape(_H, 1), b2.reshape(_H, 1)
    b3c = b3.reshape(3, 1)

    arb = pltpu.CompilerParams(
        dimension_semantics=("arbitrary", "arbitrary"))

    full = lambda shape: pl.BlockSpec(shape, lambda bi, ti: (0,) * len(shape))
    perb = lambda shape: pl.BlockSpec(shape, lambda bi, ti: (bi,) + (0,) * (len(shape) - 1))
    tile = lambda shape: pl.BlockSpec(shape, lambda bi, ti: (bi, 0, ti))

    idx, wn = pl.pallas_call(
        _knn_body,
        grid=(_B, _NT),
        in_specs=[perb((1, _NC, 3)), perb((1, _NC, 3)), tile((1, 3, _T)),
                  tile((1, 3, _T))],
        out_specs=[tile((1, 3, _T)), tile((1, 3, _T))],
        out_shape=[jax.ShapeDtypeStruct((_B, 3, _NF), i32),
                   jax.ShapeDtypeStruct((_B, 3, _NF), f32)],
        scratch_shapes=[pltpu.VMEM((_NC, 7), f32)],
        compiler_params=arb,
    )(pct, pcb, pfb, P_fine_b3n)

    mesh = plsc.VectorSubcoreMesh(core_axis_name="c", subcore_axis_name="s")
    dp3 = functools.partial(
        pl.kernel,
        out_type=jax.ShapeDtypeStruct((_B, 3, _NF), f32),
        mesh=mesh,
        compiler_params=pltpu.CompilerParams(needs_layout_passes=False),
        scratch_types=[pltpu.VMEM((3, _PW), i32), pltpu.VMEM((3, _PW), f32),
                       pltpu.VMEM((3 * _NC,), f32), pltpu.VMEM((3, _PW), f32)],
    )(_sc_gather_body)(idx, wn, dP_coarse_b3n.reshape(_B, 3 * _NC))

    tile3 = lambda shape: pl.BlockSpec(
        shape, lambda bi, pi, ti: (bi, 0, jnp.where(pi == 1, 0, ti)))
    tile0 = lambda shape: pl.BlockSpec(
        shape, lambda bi, pi, ti: (bi, 0, jnp.where(pi == 0, ti, 0)))
    tile2 = lambda shape: pl.BlockSpec(
        shape, lambda bi, pi, ti: (bi, 0, jnp.where(pi == 2, ti, 0)))
    full3 = lambda shape: pl.BlockSpec(
        shape, lambda bi, pi, ti: (0,) * len(shape))

    out = pl.pallas_call(
        _mlp_body,
        grid=(_B, 3, _NTM),
        in_specs=[tile3((1, 3, _TM)), tile0((1, _CSKIP, _TM)),
                  full3((_H, 3)), full3((_H, _CSKIP)),
                  full3((_H, 1)), full3((_H, 1)), full3((_H, _H)),
                  full3((_H, 1)), full3((_H, 1)), full3((3, _H)),
                  full3((3, 1))],
        out_specs=tile2((1, 3, _TM)),
        out_shape=jax.ShapeDtypeStruct((_B, 3, _NF), f32),
        scratch_shapes=[pltpu.VMEM((_H, _NF), bf16), pltpu.VMEM((_H, _NF), f32),
                        pltpu.VMEM((_H, 1), f32), pltpu.VMEM((_H, 1), f32),
                        pltpu.VMEM((_H, 1), f32), pltpu.VMEM((_H, 1), f32)],
        compiler_params=pltpu.CompilerParams(
            dimension_semantics=("arbitrary", "arbitrary", "arbitrary")),
    )(dp3, fsb, w1a, w1b, g1c, b1c, W2, g2c, b2c, W3, b3c)

    return out
